# baseline (speedup 1.0000x reference)
import functools

import jax
import jax.numpy as jnp
from jax import lax
from jax.experimental import pallas as pl
from jax.experimental.pallas import tpu as pltpu
from jax.experimental.pallas import tpu_sc as plsc

NC = 2
NS = 16
CH = 80
K = 8
NBUF = 4
LA = NBUF - 1


def _sc_aggregate(n_pad, ns, x, eidx, zrows):
    mesh = plsc.VectorSubcoreMesh(
        core_axis_name="c", subcore_axis_name="s", num_cores=NC, num_subcores=NS
    )
    rpz = n_pad // NS
    iters = ns * K

    @functools.partial(
        pl.kernel,
        out_type=[
            jax.ShapeDtypeStruct((NC, n_pad, 128), jnp.float32),
            jax.ShapeDtypeStruct((NC * n_pad,), jnp.float32),
        ],
        mesh=mesh,
        scratch_types=[
            pltpu.VMEM((2, K, 2, CH), jnp.int32),
            pltpu.VMEM((NBUF, CH, 128), jnp.float32),
            pltpu.VMEM((CH,), jnp.float32),
            pltpu.VMEM((n_pad // NS,), jnp.float32),
            [pltpu.SemaphoreType.DMA] * 2,
            [pltpu.SemaphoreType.DMA] * NBUF,
            [pltpu.SemaphoreType.DMA] * NBUF,
            pltpu.VMEM_SHARED((n_pad, 128), jnp.float32),
            pltpu.VMEM_SHARED((n_pad,), jnp.float32),
        ],
    )
    def agg(x_hbm, eidx_hbm, zrows_hbm, part_hbm, degp_hbm,
            idx_v, rows_v, ones_v, deg_v, isems, rsems, ssems,
            agg_sh, deg_sh):
        cid = lax.axis_index("c")
        sid = lax.axis_index("s")
        t = cid * NS + sid

        pltpu.async_copy(eidx_hbm.at[t, 0], idx_v.at[0], isems[0])

        pltpu.sync_copy(zrows_hbm.at[pl.ds(sid * rpz, rpz)],
                        agg_sh.at[pl.ds(sid * rpz, rpz)])
        for i in range(rpz // 16):
            deg_v[pl.ds(i * 16, 16)] = jnp.zeros((16,), jnp.float32)
        pltpu.sync_copy(deg_v, deg_sh.at[pl.ds(sid * rpz, rpz)])
        for i in range(CH // 16):
            ones_v[pl.ds(i * 16, 16)] = jnp.ones((16,), jnp.float32)

        plsc.subcore_barrier()

        pltpu.make_async_copy(eidx_hbm.at[t, 0], idx_v.at[0], isems[0]).wait()
        for k in range(LA):
            pltpu.async_copy(x_hbm.at[idx_v.at[0, k, 0]], rows_v.at[k],
                             rsems[k])

        def drain_scatter(sb, k, bx):
            pltpu.make_async_copy(
                rows_v.at[bx], agg_sh.at[idx_v.at[sb, k, 1]],
                ssems[bx]).wait()
            pltpu.make_async_copy(
                ones_v, deg_sh.at[idx_v.at[sb, k, 1]], ssems[bx]).wait()

        def super_step(s, sb):
            sb1 = 1 - sb
            for k in range(K):
                b = k % NBUF
                b2 = (k + LA) % NBUF
                pltpu.make_async_copy(
                    x_hbm.at[idx_v.at[sb, k, 0]], rows_v.at[b], rsems[b]).wait()

                if k == 0:
                    @pl.when(s > 0)
                    def _():
                        drain_scatter(sb, k, b2)
                elif k == 1:
                    @pl.when(s + 1 < ns)
                    def _():
                        pltpu.async_copy(eidx_hbm.at[t, s + 1], idx_v.at[sb1],
                                         isems[sb1])
                    drain_scatter(sb, k, b2)
                else:
                    drain_scatter(sb, k, b2)

                if k < K - LA:
                    pltpu.async_copy(
                        x_hbm.at[idx_v.at[sb, k + LA, 0]], rows_v.at[b2],
                        rsems[b2])
                elif k == K - LA:
                    @pl.when(s + 1 < ns)
                    def _():
                        pltpu.make_async_copy(
                            eidx_hbm.at[t, s + 1], idx_v.at[sb1],
                            isems[sb1]).wait()
                        pltpu.async_copy(
                            x_hbm.at[idx_v.at[sb1, 0, 0]], rows_v.at[b2],
                            rsems[b2])
                else:
                    nk = k - (K - LA)

                    @pl.when(s + 1 < ns)
                    def _():
                        pltpu.async_copy(
                            x_hbm.at[idx_v.at[sb1, nk, 0]], rows_v.at[b2],
                            rsems[b2])

                pltpu.async_copy(rows_v.at[b], agg_sh.at[idx_v.at[sb, k, 1]],
                                 ssems[b], add=True)
                pltpu.async_copy(ones_v, deg_sh.at[idx_v.at[sb, k, 1]],
                                 ssems[b], add=True)

        def outer(s0, carry):
            super_step(s0 * 2, 0)
            super_step(s0 * 2 + 1, 1)
            return carry

        lax.fori_loop(0, ns // 2, outer, 0)

        drain_scatter(1, K - 1, (K - 1) % NBUF)

        plsc.subcore_barrier()

        pltpu.sync_copy(agg_sh.at[pl.ds(sid * rpz, rpz)],
                        part_hbm.at[cid, pl.ds(sid * rpz, rpz)])
        pltpu.sync_copy(deg_sh.at[pl.ds(sid * rpz, rpz)], deg_v)
        pltpu.sync_copy(deg_v,
                        degp_hbm.at[pl.ds(cid * n_pad + sid * rpz, rpz)])

    return agg(x, eidx, zrows)


def _tc_combine_body(n, x_ref, p_ref, dp_ref, c00_ref, c01t_ref, c10_ref,
                     c11_ref, out_ref):
    c00 = c00_ref[...]
    c01t = c01t_ref[...]
    c10 = c10_ref[...]
    c11 = c11_ref[...]
    w0 = c00[:, 0:1] * c10 + c01t[0:1, :] * c11
    w1 = c00[:, 1:2] * c10 + c01t[1:2, :] * c11
    p = (p_ref[0] + p_ref[1])[:n]
    d = (dp_ref[0] + dp_ref[1])[:n]
    nm = p * (1.0 / jnp.maximum(d, 1.0))
    out_ref[...] = (
        jnp.dot(x_ref[...], w0, preferred_element_type=jnp.float32)
        + jnp.dot(nm, w1, preferred_element_type=jnp.float32)
    )


def kernel(x, edge_index, batch, coeffs00, coeffs01, coeffs10, coeffs11):
    n, in_ch = x.shape
    e = edge_index.shape[1]
    tiles = NC * NS
    ns = -(-e // (tiles * CH * K))
    ns += ns % 2
    iters = ns * K
    e_pad = tiles * iters * CH
    n_pad = -(-(n + 1) // 256) * 256

    pad_pos = jnp.arange(e_pad - e, dtype=jnp.int32)
    src = jnp.concatenate([edge_index[0], pad_pos % n])
    dst = jnp.concatenate([edge_index[1], n + pad_pos % (n_pad - n)])
    eidx = jnp.stack(
        [src.reshape(tiles, ns, K, CH), dst.reshape(tiles, ns, K, CH)], axis=3)
    zrows = jnp.zeros((n_pad, in_ch), jnp.float32)

    parts, degp = _sc_aggregate(n_pad, ns, x, eidx, zrows)

    out = pl.pallas_call(
        functools.partial(_tc_combine_body, n),
        out_shape=jax.ShapeDtypeStruct((n, in_ch), jnp.float32),
    )(x, parts, degp.reshape(NC, n_pad, 1), coeffs00, coeffs01.T, coeffs10,
      coeffs11)
    return out


# speedup vs baseline: 1.0959x; 1.0959x over previous
import functools

import jax
import jax.numpy as jnp
from jax import lax
from jax.experimental import pallas as pl
from jax.experimental.pallas import tpu as pltpu
from jax.experimental.pallas import tpu_sc as plsc

NC = 2
NS = 16
CH = 80
K = 8
NBUF = 4
LA = NBUF - 1


def _sc_aggregate(n_pad, ns, x, eidx, zrows):
    mesh = plsc.VectorSubcoreMesh(
        core_axis_name="c", subcore_axis_name="s", num_cores=NC, num_subcores=NS
    )
    rpz = n_pad // NS
    iters = ns * K

    @functools.partial(
        pl.kernel,
        out_type=[
            jax.ShapeDtypeStruct((NC, n_pad, 128), jnp.float32),
            jax.ShapeDtypeStruct((NC * n_pad,), jnp.float32),
        ],
        mesh=mesh,
        scratch_types=[
            pltpu.VMEM((2, K, 2, CH), jnp.int32),
            pltpu.VMEM((NBUF, CH, 128), jnp.float32),
            pltpu.VMEM((CH,), jnp.float32),
            pltpu.VMEM((n_pad // NS,), jnp.float32),
            [pltpu.SemaphoreType.DMA] * 2,
            [pltpu.SemaphoreType.DMA] * NBUF,
            [pltpu.SemaphoreType.DMA] * NBUF,
            pltpu.VMEM_SHARED((n_pad, 128), jnp.float32),
            pltpu.VMEM_SHARED((n_pad,), jnp.float32),
        ],
    )
    def agg(x_hbm, eidx_hbm, zrows_hbm, part_hbm, degp_hbm,
            idx_v, rows_v, ones_v, deg_v, isems, rsems, ssems,
            agg_sh, deg_sh):
        cid = lax.axis_index("c")
        sid = lax.axis_index("s")
        t = cid * NS + sid

        pltpu.async_copy(eidx_hbm.at[t, 0], idx_v.at[0], isems[0])

        pltpu.sync_copy(zrows_hbm.at[pl.ds(sid * rpz, rpz)],
                        agg_sh.at[pl.ds(sid * rpz, rpz)])
        for i in range(rpz // 16):
            deg_v[pl.ds(i * 16, 16)] = jnp.zeros((16,), jnp.float32)
        pltpu.sync_copy(deg_v, deg_sh.at[pl.ds(sid * rpz, rpz)])
        for i in range(CH // 16):
            ones_v[pl.ds(i * 16, 16)] = jnp.ones((16,), jnp.float32)

        plsc.subcore_barrier()

        pltpu.make_async_copy(eidx_hbm.at[t, 0], idx_v.at[0], isems[0]).wait()
        for k in range(LA):
            pltpu.async_copy(x_hbm.at[idx_v.at[0, k, 0]], rows_v.at[k],
                             rsems[k])

        def drain_scatter(sb, k, bx):
            pltpu.make_async_copy(
                rows_v.at[bx], agg_sh.at[idx_v.at[sb, k, 1]],
                ssems[bx]).wait()
            pltpu.make_async_copy(
                ones_v, deg_sh.at[idx_v.at[sb, k, 1]], ssems[bx]).wait()

        def super_step(s, sb):
            sb1 = 1 - sb
            for k in range(K):
                b = k % NBUF
                b2 = (k + LA) % NBUF
                pltpu.make_async_copy(
                    x_hbm.at[idx_v.at[sb, k, 0]], rows_v.at[b], rsems[b]).wait()

                if k == 0:
                    @pl.when(s > 0)
                    def _():
                        drain_scatter(sb, k, b2)
                elif k == 1:
                    @pl.when(s + 1 < ns)
                    def _():
                        pltpu.async_copy(eidx_hbm.at[t, s + 1], idx_v.at[sb1],
                                         isems[sb1])
                    drain_scatter(sb, k, b2)
                else:
                    drain_scatter(sb, k, b2)

                if k < K - LA:
                    pltpu.async_copy(
                        x_hbm.at[idx_v.at[sb, k + LA, 0]], rows_v.at[b2],
                        rsems[b2])
                elif k == K - LA:
                    @pl.when(s + 1 < ns)
                    def _():
                        pltpu.make_async_copy(
                            eidx_hbm.at[t, s + 1], idx_v.at[sb1],
                            isems[sb1]).wait()
                        pltpu.async_copy(
                            x_hbm.at[idx_v.at[sb1, 0, 0]], rows_v.at[b2],
                            rsems[b2])
                else:
                    nk = k - (K - LA)

                    @pl.when(s + 1 < ns)
                    def _():
                        pltpu.async_copy(
                            x_hbm.at[idx_v.at[sb1, nk, 0]], rows_v.at[b2],
                            rsems[b2])

                pltpu.async_copy(rows_v.at[b], agg_sh.at[idx_v.at[sb, k, 1]],
                                 ssems[b], add=True)
                pltpu.async_copy(ones_v, deg_sh.at[idx_v.at[sb, k, 1]],
                                 ssems[b], add=True)

        def outer(s0, carry):
            super_step(s0 * 2, 0)
            super_step(s0 * 2 + 1, 1)
            return carry

        lax.fori_loop(0, ns // 2, outer, 0)

        drain_scatter(1, K - 1, (K - 1) % NBUF)

        plsc.subcore_barrier()

        pltpu.sync_copy(agg_sh.at[pl.ds(sid * rpz, rpz)],
                        part_hbm.at[cid, pl.ds(sid * rpz, rpz)])
        pltpu.sync_copy(deg_sh.at[pl.ds(sid * rpz, rpz)], deg_v)
        pltpu.sync_copy(deg_v,
                        degp_hbm.at[pl.ds(cid * n_pad + sid * rpz, rpz)])

    return agg(x, eidx, zrows)


def _tc_combine_body(n, x_ref, p_ref, dp_ref, c00_ref, c01t_ref, c10_ref,
                     c11_ref, out_ref):
    c00 = c00_ref[...]
    c01t = c01t_ref[...]
    c10 = c10_ref[...]
    c11 = c11_ref[...]
    w0 = c00[:, 0:1] * c10 + c01t[0:1, :] * c11
    w1 = c00[:, 1:2] * c10 + c01t[1:2, :] * c11
    p = (p_ref[0] + p_ref[1])[:n]
    dflat = dp_ref[...]
    n_pad = dflat.shape[0] // 2
    d = (dflat[:n_pad] + dflat[n_pad:])[:n]
    nm = p * (1.0 / jnp.maximum(d, 1.0))[:, None]
    out_ref[...] = (
        jnp.dot(x_ref[...], w0, preferred_element_type=jnp.float32)
        + jnp.dot(nm, w1, preferred_element_type=jnp.float32)
    )


def kernel(x, edge_index, batch, coeffs00, coeffs01, coeffs10, coeffs11):
    n, in_ch = x.shape
    e = edge_index.shape[1]
    tiles = NC * NS
    ns = -(-e // (tiles * CH * K))
    ns += ns % 2
    iters = ns * K
    e_pad = tiles * iters * CH
    n_pad = -(-(n + 1) // 256) * 256

    pad_pos = jnp.arange(e_pad - e, dtype=jnp.int32)
    src = jnp.concatenate([edge_index[0], pad_pos % n])
    dst = jnp.concatenate([edge_index[1], n + pad_pos % (n_pad - n)])
    eidx = jnp.stack(
        [src.reshape(tiles, ns, K, CH), dst.reshape(tiles, ns, K, CH)], axis=3)
    zrows = jnp.zeros((n_pad, in_ch), jnp.float32)

    parts, degp = _sc_aggregate(n_pad, ns, x, eidx, zrows)

    out = pl.pallas_call(
        functools.partial(_tc_combine_body, n),
        out_shape=jax.ShapeDtypeStruct((n, in_ch), jnp.float32),
    )(x, parts, degp, coeffs00, coeffs01.T, coeffs10, coeffs11)
    return out
